# Initial kernel scaffold; baseline (speedup 1.0000x reference)
import jax
import jax.numpy as jnp
from jax.experimental import pallas as pl


def kernel(x, e, snorm_n, snorm_e, edge_index, graph_ids, embed, W_gcn, b_gcn, gamma, beta, W_r1, b_r1, W_r2, b_r2):
    raise NotImplementedError("write your pallas kernel here")


# speedup vs baseline: 7.6615x; 7.6615x over previous
import functools

import jax
import jax.numpy as jnp
from jax import lax
from jax.experimental import pallas as pl
from jax.experimental.pallas import tpu as pltpu
from jax.experimental.pallas import tpu_sc as plsc

N = 10000
E = 320000
F = 128
NG = 256
NC = 2
NS = 16
NW = NC * NS
CHUNK = 128
KSTEPS = 80
EPW = KSTEPS * CHUNK
EPAD = NW * EPW
NPAD = 10240
RPT = NPAD // NS
NPD = 10112

_mesh = plsc.VectorSubcoreMesh(core_axis_name="c", subcore_axis_name="s")
_f32 = jnp.float32
_HI = lax.Precision.HIGHEST


def _edge_agg_body(m_hbm, src_hbm, dst_hbm, zeros_hbm, out_hbm,
                   src_v, dst_v, rows_v, agg_sh, sem):
    c = lax.axis_index("c")
    s = lax.axis_index("s")
    w = s * NC + c
    base = s * RPT
    pltpu.sync_copy(zeros_hbm, rows_v)
    for k in range(RPT // CHUNK):
        pltpu.sync_copy(rows_v, agg_sh.at[pl.ds(base + k * CHUNK, CHUNK)])
    pltpu.sync_copy(src_hbm.at[w], src_v)
    pltpu.sync_copy(dst_hbm.at[w], dst_v)
    plsc.subcore_barrier()

    def body(j, carry):
        pltpu.async_copy(m_hbm.at[src_v.at[j]], rows_v, sem).wait()
        pltpu.sync_copy(rows_v, agg_sh.at[dst_v.at[j]], add=True)
        return carry

    lax.fori_loop(0, KSTEPS, body, 0)
    plsc.subcore_barrier()
    for k in range(RPT // CHUNK):
        pltpu.sync_copy(agg_sh.at[pl.ds(base + k * CHUNK, CHUNK)], rows_v)
        pltpu.sync_copy(rows_v, out_hbm.at[c, pl.ds(base + k * CHUNK, CHUNK)])


def _degrees_body(src_hbm, dst_hbm, outo_hbm, outi_hbm,
                  src_v, dst_v, dego_v, degi_v):
    c = lax.axis_index("c")
    s = lax.axis_index("s")
    w = s * NC + c
    pltpu.sync_copy(src_hbm.at[w], src_v)
    pltpu.sync_copy(dst_hbm.at[w], dst_v)
    zero16 = jnp.zeros((16,), _f32)

    def zbody(i, carry):
        dego_v[pl.ds(i * 16, 16)] = zero16
        degi_v[pl.ds(i * 16, 16)] = zero16
        return carry

    lax.fori_loop(0, NPD // 16, zbody, 0)
    ones16 = jnp.ones((16,), _f32)

    def body(i, carry):
        j = i // (CHUNK // 16)
        k = i % (CHUNK // 16)
        so = src_v[j, pl.ds(k * 16, 16)]
        si = dst_v[j, pl.ds(k * 16, 16)]
        plsc.addupdate_scatter(dego_v, [so], ones16)
        plsc.addupdate_scatter(degi_v, [si], ones16)
        return carry

    lax.fori_loop(0, KSTEPS * (CHUNK // 16), body, 0)
    pltpu.sync_copy(dego_v, outo_hbm.at[w])
    pltpu.sync_copy(degi_v, outi_hbm.at[w])


_AGG_SCRATCH = [
    pltpu.VMEM((KSTEPS, CHUNK), jnp.int32),
    pltpu.VMEM((KSTEPS, CHUNK), jnp.int32),
    pltpu.VMEM((CHUNK, F), _f32),
    pltpu.VMEM_SHARED((NPAD, F), _f32),
    pltpu.SemaphoreType.DMA,
]
_DEG_SCRATCH = [
    pltpu.VMEM((KSTEPS, CHUNK), jnp.int32),
    pltpu.VMEM((KSTEPS, CHUNK), jnp.int32),
    pltpu.VMEM((NPD,), _f32),
    pltpu.VMEM((NPD,), _f32),
]

_edge_agg = pl.kernel(
    _edge_agg_body,
    out_type=jax.ShapeDtypeStruct((NC, NPAD, F), _f32),
    mesh=_mesh,
    scratch_types=_AGG_SCRATCH,
)

_degrees = pl.kernel(
    _degrees_body,
    out_type=(jax.ShapeDtypeStruct((NW, NPD), _f32),
              jax.ShapeDtypeStruct((NW, NPD), _f32)),
    mesh=_mesh,
    scratch_types=_DEG_SCRATCH,
    compiler_params=pltpu.CompilerParams(needs_layout_passes=False),
)


def _prep_body(x_ref, embed_ref, dego_ref, degi_ref, m0_ref, invi_ref, invo_ref):
    dego = jnp.sum(dego_ref[...][:, :N], axis=0)
    degi = jnp.sum(degi_ref[...][:, :N], axis=0)
    invo = jnp.where(dego > 0, 1.0 / jnp.sqrt(jnp.maximum(dego, 1.0)), 0.0)[:, None]
    invi = jnp.where(degi > 0, 1.0 / jnp.sqrt(jnp.maximum(degi, 1.0)), 0.0)[:, None]
    onehot = (x_ref[...] == lax.broadcasted_iota(jnp.int32, (N, 28), 1)).astype(_f32)
    h0 = jnp.dot(onehot, embed_ref[...], preferred_element_type=_f32, precision=_HI)
    m0_ref[...] = h0 * invo
    invi_ref[...] = invi
    invo_ref[...] = invo


_prep = pl.pallas_call(
    _prep_body,
    out_shape=(jax.ShapeDtypeStruct((N, F), _f32),
               jax.ShapeDtypeStruct((N, 1), _f32),
               jax.ShapeDtypeStruct((N, 1), _f32)),
)


def _bf16_dot(a, b):
    return jnp.dot(a.astype(jnp.bfloat16), b.astype(jnp.bfloat16),
                   preferred_element_type=_f32)


def _norm_tanh(aggp_ref, invi_ref, snorm_ref, w_ref, b_ref, g_ref, be_ref):
    agg = (aggp_ref[0, :N, :] + aggp_ref[1, :N, :]) * invi_ref[...]
    h = _bf16_dot(agg, w_ref[...]) + b_ref[...]
    h = h * snorm_ref[...]
    mu = jnp.mean(h, axis=0, keepdims=True)
    var = jnp.mean((h - mu) ** 2, axis=0, keepdims=True)
    h = g_ref[...] * (h - mu) / jnp.sqrt(var + 1e-5) + be_ref[...]
    return jnp.tanh(h)


def _layer_body(aggp_ref, invi_ref, invo_ref, snorm_ref, w_ref, b_ref,
                g_ref, be_ref, m_ref):
    h = _norm_tanh(aggp_ref, invi_ref, snorm_ref, w_ref, b_ref, g_ref, be_ref)
    m_ref[...] = h * invo_ref[...]


_layer = pl.pallas_call(
    _layer_body,
    out_shape=jax.ShapeDtypeStruct((N, F), _f32),
)


def _final_body(aggp_ref, invi_ref, snorm_ref, w_ref, b_ref, g_ref, be_ref,
                gid_ref, wr1_ref, br1_ref, wr2_ref, br2_ref, out_ref):
    h = _norm_tanh(aggp_ref, invi_ref, snorm_ref, w_ref, b_ref, g_ref, be_ref)
    onehot_t = (gid_ref[...] == lax.broadcasted_iota(jnp.int32, (NG, N), 0)).astype(_f32)
    sums = jnp.dot(onehot_t, h, preferred_element_type=_f32, precision=_HI)
    cnts = jnp.sum(onehot_t, axis=1)
    hg = sums / jnp.maximum(cnts, 1.0)[:, None]
    hg = jnp.maximum(hg, 0.0)
    hg = jnp.maximum(_bf16_dot(hg, wr1_ref[...]) + br1_ref[...], 0.0)
    out_ref[...] = _bf16_dot(hg, wr2_ref[...]) + br2_ref[...]


_final = pl.pallas_call(
    _final_body,
    out_shape=jax.ShapeDtypeStruct((NG, 1), _f32),
)


def kernel(x, e, snorm_n, snorm_e, edge_index, graph_ids, embed, W_gcn,
           b_gcn, gamma, beta, W_r1, b_r1, W_r2, b_r2):
    src = edge_index[0].astype(jnp.int32)
    dst = edge_index[1].astype(jnp.int32)
    pad = EPAD - E
    ar = jnp.arange(pad, dtype=jnp.int32)
    src_a = jnp.concatenate([src, ar % N]).reshape(NW, KSTEPS, CHUNK)
    dst_a = jnp.concatenate([dst, N + ar % (NPAD - N)]).reshape(NW, KSTEPS, CHUNK)
    src_d = jnp.concatenate([src, N + ar % (NPD - N)]).reshape(NW, KSTEPS, CHUNK)
    dst_d = jnp.concatenate([dst, N + ar % (NPD - N)]).reshape(NW, KSTEPS, CHUNK)
    zeros128 = jnp.zeros((CHUNK, F), _f32)

    dego_p, degi_p = _degrees(src_d, dst_d)
    x2 = x.astype(jnp.int32).reshape(N, 1)
    m, invi, invo = _prep(x2, embed, dego_p, degi_p)

    snorm = snorm_n.astype(_f32)
    for i in range(4):
        aggp = _edge_agg(m, src_a, dst_a, zeros128)
        m = _layer(aggp, invi, invo, snorm, W_gcn[i], b_gcn[i].reshape(1, F),
                   gamma[i].reshape(1, F), beta[i].reshape(1, F))
    aggp = _edge_agg(m, src_a, dst_a, zeros128)
    gid = graph_ids.astype(jnp.int32).reshape(1, N)
    return _final(aggp, invi, snorm, W_gcn[4], b_gcn[4].reshape(1, F),
                  gamma[4].reshape(1, F), beta[4].reshape(1, F), gid,
                  W_r1, b_r1.reshape(1, F // 2), W_r2, b_r2.reshape(1, 1))


# speedup vs baseline: 9.1136x; 1.1895x over previous
import functools

import jax
import jax.numpy as jnp
from jax import lax
from jax.experimental import pallas as pl
from jax.experimental.pallas import tpu as pltpu
from jax.experimental.pallas import tpu_sc as plsc

N = 10000
E = 320000
F = 128
NG = 256
NC = 2
NS = 16
NW = NC * NS
CHUNK = 128
KSTEPS = 80
EPW = KSTEPS * CHUNK
EPAD = NW * EPW
NPAD = 10240
RPT = NPAD // NS
NPD = 10112
IBLK = 8

_mesh = plsc.VectorSubcoreMesh(core_axis_name="c", subcore_axis_name="s")
_f32 = jnp.float32
_HI = lax.Precision.HIGHEST


def _edge_agg_body(m_hbm, src_hbm, dst_hbm, zeros_hbm, out_hbm,
                   src_v, dst_v, rows0_v, rows1_v, agg_sh, sem):
    c = lax.axis_index("c")
    s = lax.axis_index("s")
    w = s * NC + c
    base = s * RPT
    pltpu.sync_copy(zeros_hbm, rows0_v)
    for k in range(RPT // CHUNK):
        pltpu.sync_copy(rows0_v, agg_sh.at[pl.ds(base + k * CHUNK, CHUNK)])
    plsc.subcore_barrier()

    def blk(b, carry):
        pltpu.sync_copy(src_hbm.at[w, pl.ds(b * IBLK, IBLK)], src_v)
        pltpu.sync_copy(dst_hbm.at[w, pl.ds(b * IBLK, IBLK)], dst_v)
        pltpu.async_copy(m_hbm.at[src_v.at[0]], rows0_v, sem)

        def body(jj, c2):
            j = jj * 2
            pltpu.make_async_copy(m_hbm.at[src_v.at[j]], rows0_v, sem).wait()
            pltpu.async_copy(m_hbm.at[src_v.at[j + 1]], rows1_v, sem)
            pltpu.sync_copy(rows0_v, agg_sh.at[dst_v.at[j]], add=True)
            pltpu.make_async_copy(m_hbm.at[src_v.at[j + 1]], rows1_v, sem).wait()

            @pl.when(jj + 1 < IBLK // 2)
            def _():
                pltpu.async_copy(m_hbm.at[src_v.at[j + 2]], rows0_v, sem)

            pltpu.sync_copy(rows1_v, agg_sh.at[dst_v.at[j + 1]], add=True)
            return c2

        lax.fori_loop(0, IBLK // 2, body, 0)
        return carry

    lax.fori_loop(0, KSTEPS // IBLK, blk, 0)
    plsc.subcore_barrier()
    for k in range(RPT // CHUNK):
        pltpu.sync_copy(agg_sh.at[pl.ds(base + k * CHUNK, CHUNK)], rows0_v)
        pltpu.sync_copy(rows0_v, out_hbm.at[c, pl.ds(base + k * CHUNK, CHUNK)])


def _degrees_body(src_hbm, dst_hbm, outo_hbm, outi_hbm,
                  src_v, dst_v, dego_v, degi_v):
    c = lax.axis_index("c")
    s = lax.axis_index("s")
    w = s * NC + c
    pltpu.sync_copy(src_hbm.at[w], src_v)
    pltpu.sync_copy(dst_hbm.at[w], dst_v)
    zero16 = jnp.zeros((16,), _f32)

    def zbody(i, carry):
        dego_v[pl.ds(i * 16, 16)] = zero16
        degi_v[pl.ds(i * 16, 16)] = zero16
        return carry

    lax.fori_loop(0, NPD // 16, zbody, 0)
    ones16 = jnp.ones((16,), _f32)

    def body(i, carry):
        j = i // (CHUNK // 16)
        k = i % (CHUNK // 16)
        so = src_v[j, pl.ds(k * 16, 16)]
        si = dst_v[j, pl.ds(k * 16, 16)]
        plsc.addupdate_scatter(dego_v, [so], ones16)
        plsc.addupdate_scatter(degi_v, [si], ones16)
        return carry

    lax.fori_loop(0, KSTEPS * (CHUNK // 16), body, 0)
    pltpu.sync_copy(dego_v, outo_hbm.at[w])
    pltpu.sync_copy(degi_v, outi_hbm.at[w])


_AGG_SCRATCH = [
    pltpu.VMEM((IBLK, CHUNK), jnp.int32),
    pltpu.VMEM((IBLK, CHUNK), jnp.int32),
    pltpu.VMEM((CHUNK, F), _f32),
    pltpu.VMEM((CHUNK, F), _f32),
    pltpu.VMEM_SHARED((NPAD, F), _f32),
    pltpu.SemaphoreType.DMA,
]
_DEG_SCRATCH = [
    pltpu.VMEM((KSTEPS, CHUNK), jnp.int32),
    pltpu.VMEM((KSTEPS, CHUNK), jnp.int32),
    pltpu.VMEM((NPD,), _f32),
    pltpu.VMEM((NPD,), _f32),
]

_edge_agg = pl.kernel(
    _edge_agg_body,
    out_type=jax.ShapeDtypeStruct((NC, NPAD, F), _f32),
    mesh=_mesh,
    scratch_types=_AGG_SCRATCH,
)

_degrees = pl.kernel(
    _degrees_body,
    out_type=(jax.ShapeDtypeStruct((NW, NPD), _f32),
              jax.ShapeDtypeStruct((NW, NPD), _f32)),
    mesh=_mesh,
    scratch_types=_DEG_SCRATCH,
    compiler_params=pltpu.CompilerParams(needs_layout_passes=False),
)


def _prep_body(x_ref, embed_ref, dego_ref, degi_ref, m0_ref, invi_ref, invo_ref):
    dego = jnp.sum(dego_ref[...][:, :N], axis=0)
    degi = jnp.sum(degi_ref[...][:, :N], axis=0)
    invo = jnp.where(dego > 0, 1.0 / jnp.sqrt(jnp.maximum(dego, 1.0)), 0.0)[:, None]
    invi = jnp.where(degi > 0, 1.0 / jnp.sqrt(jnp.maximum(degi, 1.0)), 0.0)[:, None]
    onehot = (x_ref[...] == lax.broadcasted_iota(jnp.int32, (N, 28), 1)).astype(_f32)
    h0 = jnp.dot(onehot, embed_ref[...], preferred_element_type=_f32, precision=_HI)
    m0_ref[...] = h0 * invo
    invi_ref[...] = invi
    invo_ref[...] = invo


_prep = pl.pallas_call(
    _prep_body,
    out_shape=(jax.ShapeDtypeStruct((N, F), _f32),
               jax.ShapeDtypeStruct((N, 1), _f32),
               jax.ShapeDtypeStruct((N, 1), _f32)),
)


def _bf16_dot(a, b):
    return jnp.dot(a.astype(jnp.bfloat16), b.astype(jnp.bfloat16),
                   preferred_element_type=_f32)


def _norm_tanh(aggp_ref, invi_ref, snorm_ref, w_ref, b_ref, g_ref, be_ref):
    agg = (aggp_ref[0, :N, :] + aggp_ref[1, :N, :]) * invi_ref[...]
    h = _bf16_dot(agg, w_ref[...]) + b_ref[...]
    h = h * snorm_ref[...]
    mu = jnp.mean(h, axis=0, keepdims=True)
    var = jnp.mean((h - mu) ** 2, axis=0, keepdims=True)
    h = g_ref[...] * (h - mu) / jnp.sqrt(var + 1e-5) + be_ref[...]
    return jnp.tanh(h)


def _layer_body(aggp_ref, invi_ref, invo_ref, snorm_ref, w_ref, b_ref,
                g_ref, be_ref, m_ref):
    h = _norm_tanh(aggp_ref, invi_ref, snorm_ref, w_ref, b_ref, g_ref, be_ref)
    m_ref[...] = h * invo_ref[...]


_layer = pl.pallas_call(
    _layer_body,
    out_shape=jax.ShapeDtypeStruct((N, F), _f32),
)


def _final_body(aggp_ref, invi_ref, snorm_ref, w_ref, b_ref, g_ref, be_ref,
                gid_ref, wr1_ref, br1_ref, wr2_ref, br2_ref, out_ref):
    h = _norm_tanh(aggp_ref, invi_ref, snorm_ref, w_ref, b_ref, g_ref, be_ref)
    onehot_t = (gid_ref[...] == lax.broadcasted_iota(jnp.int32, (NG, N), 0)).astype(_f32)
    sums = jnp.dot(onehot_t, h, preferred_element_type=_f32, precision=_HI)
    cnts = jnp.sum(onehot_t, axis=1)
    hg = sums / jnp.maximum(cnts, 1.0)[:, None]
    hg = jnp.maximum(hg, 0.0)
    hg = jnp.maximum(_bf16_dot(hg, wr1_ref[...]) + br1_ref[...], 0.0)
    out_ref[...] = _bf16_dot(hg, wr2_ref[...]) + br2_ref[...]


_final = pl.pallas_call(
    _final_body,
    out_shape=jax.ShapeDtypeStruct((NG, 1), _f32),
)


def kernel(x, e, snorm_n, snorm_e, edge_index, graph_ids, embed, W_gcn,
           b_gcn, gamma, beta, W_r1, b_r1, W_r2, b_r2):
    src = edge_index[0].astype(jnp.int32)
    dst = edge_index[1].astype(jnp.int32)
    pad = EPAD - E
    ar = jnp.arange(pad, dtype=jnp.int32)
    src_a = jnp.concatenate([src, ar % N]).reshape(NW, KSTEPS, CHUNK)
    dst_a = jnp.concatenate([dst, N + ar % (NPAD - N)]).reshape(NW, KSTEPS, CHUNK)
    src_d = jnp.concatenate([src, N + ar % (NPD - N)]).reshape(NW, KSTEPS, CHUNK)
    dst_d = jnp.concatenate([dst, N + ar % (NPD - N)]).reshape(NW, KSTEPS, CHUNK)
    zeros128 = jnp.zeros((CHUNK, F), _f32)

    dego_p, degi_p = _degrees(src_d, dst_d)
    x2 = x.astype(jnp.int32).reshape(N, 1)
    m, invi, invo = _prep(x2, embed, dego_p, degi_p)

    snorm = snorm_n.astype(_f32)
    for i in range(4):
        aggp = _edge_agg(m, src_a, dst_a, zeros128)
        m = _layer(aggp, invi, invo, snorm, W_gcn[i], b_gcn[i].reshape(1, F),
                   gamma[i].reshape(1, F), beta[i].reshape(1, F))
    aggp = _edge_agg(m, src_a, dst_a, zeros128)
    gid = graph_ids.astype(jnp.int32).reshape(1, N)
    return _final(aggp, invi, snorm, W_gcn[4], b_gcn[4].reshape(1, F),
                  gamma[4].reshape(1, F), beta[4].reshape(1, F), gid,
                  W_r1, b_r1.reshape(1, F // 2), W_r2, b_r2.reshape(1, 1))
